# Initial kernel scaffold; baseline (speedup 1.0000x reference)
import jax
import jax.numpy as jnp
from jax.experimental import pallas as pl


def kernel(act, W_loc, b_loc, W_conf, b_conf, dboxes_cxywh):
    raise NotImplementedError("write your pallas kernel here")


# speedup vs baseline: 922.6769x; 922.6769x over previous
import jax
import jax.numpy as jnp
from jax import lax
from jax.experimental import pallas as pl
from jax.experimental.pallas import tpu as pltpu
from jax.experimental.pallas import tpu_sc as plsc

H = 64
W_DIM = 64
P = H * W_DIM
NSZ = 6
C_LOC = 4 * NSZ
C_CONF = 21 * NSZ
C_OUT = C_LOC + C_CONF
NCLS = 21
PADL = 72
XCOLS = 4352
N_ANCH = NSZ * P
SCALE_XY = 0.1
SCALE_WH = 0.2
NMS_THR = 0.1
MAX_DETS = 100

NSUB = 16
NPER = N_ANCH // NSUB
NCHUNK = NPER // 16
NEG = float("-inf")


def _perm(v, idx):
    return v.at[idx].get(mode="promise_in_bounds")


def _bfly_max(v, lane):
    for sh in (8, 4, 2, 1):
        v = jnp.maximum(v, _perm(v, lane ^ sh))
    return v


def _bfly_min_i32(v, lane):
    for sh in (8, 4, 2, 1):
        v = jnp.minimum(v, _perm(v, lane ^ sh))
    return v


def _conv_decode_body(xpad_ref, w9_ref, bias_ref, db_ref, out_ref):
    wcol = lax.broadcasted_iota(jnp.int32, (1, P), 1) % W_DIM
    mask_l = (wcol > 0).astype(jnp.float32)
    mask_r = (wcol < W_DIM - 1).astype(jnp.float32)
    acc = None
    for dy in range(3):
        for dx in range(3):
            k = dy * 3 + dx
            off = PADL + (dy - 1) * W_DIM + (dx - 1)
            xs = xpad_ref[:, off:off + P]
            prod = jnp.dot(w9_ref[k], xs, preferred_element_type=jnp.float32)
            if dx == 0:
                prod = prod * mask_l
            elif dx == 2:
                prod = prod * mask_r
            acc = prod if acc is None else acc + prod
    acc = acc + bias_ref[...]
    for s in range(NSZ):
        lx = acc[0 + s:1 + s]
        ly = acc[6 + s:7 + s]
        lw = acc[12 + s:13 + s]
        lh = acc[18 + s:19 + s]
        dcx = db_ref[0, s:s + 1]
        dcy = db_ref[1, s:s + 1]
        dwd = db_ref[2, s:s + 1]
        dht = db_ref[3, s:s + 1]
        wv = jnp.exp(lw * SCALE_WH) * dcx
        hv = jnp.exp(lh * SCALE_WH) * dcy
        cx = lx * SCALE_XY * dwd + dcx
        cy = ly * SCALE_XY * dht + dcy
        x1 = cx - wv / 2
        y1 = cy - hv / 2
        x2 = cx + wv / 2
        y2 = cy + hv / 2
        m = acc[C_LOC + s:C_LOC + s + 1]
        for cls in range(1, NCLS):
            c = C_LOC + cls * NSZ + s
            m = jnp.maximum(m, acc[c:c + 1])
        area = (x2 - x1) * (y2 - y1)
        out_ref[0 * NSZ + s:0 * NSZ + s + 1, :] = m
        out_ref[1 * NSZ + s:1 * NSZ + s + 1, :] = x1
        out_ref[2 * NSZ + s:2 * NSZ + s + 1, :] = y1
        out_ref[3 * NSZ + s:3 * NSZ + s + 1, :] = x2
        out_ref[4 * NSZ + s:4 * NSZ + s + 1, :] = y2
        out_ref[5 * NSZ + s:5 * NSZ + s + 1, :] = area


def _nms_body(data_hbm, out_hbm, sc_v, x1_v, y1_v, x2_v, y2_v, ar_v,
              pub_v, tab_sh, tab_v, kept_v, flag_v):
    wid = lax.axis_index("s")
    base = wid * NPER
    pltpu.sync_copy(data_hbm.at[pl.ds(0 * N_ANCH + base, NPER)], sc_v)
    pltpu.sync_copy(data_hbm.at[pl.ds(1 * N_ANCH + base, NPER)], x1_v)
    pltpu.sync_copy(data_hbm.at[pl.ds(2 * N_ANCH + base, NPER)], y1_v)
    pltpu.sync_copy(data_hbm.at[pl.ds(3 * N_ANCH + base, NPER)], x2_v)
    pltpu.sync_copy(data_hbm.at[pl.ds(4 * N_ANCH + base, NPER)], y2_v)
    pltpu.sync_copy(data_hbm.at[pl.ds(5 * N_ANCH + base, NPER)], ar_v)
    lane = lax.broadcasted_iota(jnp.int32, (16,), 0)
    zv = jnp.zeros((16,), jnp.float32)

    @pl.when(wid == 0)
    def _zero_out():
        for r in range(MAX_DETS):
            kept_v[pl.ds(r * 16, 16)] = zv

    def body(it, carry):
        vbest = jnp.full((16,), NEG, jnp.float32)
        vcid = jnp.zeros((16,), jnp.int32)
        for ch in range(NCHUNK):
            svec = sc_v[pl.ds(ch * 16, 16)]
            upd = svec > vbest
            vbest = jnp.where(upd, svec, vbest)
            vcid = jnp.where(upd, ch, vcid)
        mv = _bfly_max(vbest, lane)
        m = mv[0]
        cand = jnp.where(vbest == mv, vcid * 16 + lane, jnp.int32(2 ** 30))
        liv = _bfly_min_i32(cand, lane)
        li = liv[0]
        gi = (base + li).astype(jnp.float32)
        lst = (li // 16) * 16
        lln = liv - lst
        bx1 = _perm(x1_v[pl.ds(lst, 16)], lln)[0]
        by1 = _perm(y1_v[pl.ds(lst, 16)], lln)[0]
        bx2 = _perm(x2_v[pl.ds(lst, 16)], lln)[0]
        by2 = _perm(y2_v[pl.ds(lst, 16)], lln)[0]
        bar = _perm(ar_v[pl.ds(lst, 16)], lln)[0]
        row = jnp.where(lane == 0, m, 0.0)
        row = jnp.where(lane == 1, gi, row)
        row = jnp.where(lane == 2, bx1, row)
        row = jnp.where(lane == 3, by1, row)
        row = jnp.where(lane == 4, bx2, row)
        row = jnp.where(lane == 5, by2, row)
        row = jnp.where(lane == 6, bar, row)
        pub_v[...] = row
        pltpu.sync_copy(pub_v, tab_sh.at[pl.ds(wid * 16, 16)])
        plsc.subcore_barrier()
        pltpu.sync_copy(tab_sh, tab_v)
        plsc.subcore_barrier()
        r0 = tab_v[pl.ds(0, 16)]
        bs, bi = r0[0], r0[1]
        wrow = r0
        for w in range(1, NSUB):
            rw = tab_v[pl.ds(w * 16, 16)]
            s_w, i_w = rw[0], rw[1]
            better = (s_w > bs) | ((s_w == bs) & (i_w < bi))
            bs = jnp.where(better, s_w, bs)
            bi = jnp.where(better, i_w, bi)
            wrow = jnp.where(better, rw, wrow)
        wx1, wy1 = wrow[2], wrow[3]
        wx2, wy2, wa = wrow[4], wrow[5], wrow[6]
        alive2 = bs > NEG

        @pl.when(alive2)
        def _suppress():
            for ch in range(NCHUNK):
                sl = pl.ds(ch * 16, 16)
                xx1 = jnp.maximum(x1_v[sl], wx1)
                yy1 = jnp.maximum(y1_v[sl], wy1)
                xx2 = jnp.minimum(x2_v[sl], wx2)
                yy2 = jnp.minimum(y2_v[sl], wy2)
                iw = jnp.maximum(xx2 - xx1, 0.0)
                ih = jnp.maximum(yy2 - yy1, 0.0)
                inter = iw * ih
                ovr = inter / (wa + ar_v[sl] - inter)
                sc_cur = sc_v[sl]
                sc_v[sl] = jnp.where(ovr > NMS_THR, NEG, sc_cur)

        @pl.when(alive2 & (wid == 0))
        def _record():
            krow = jnp.where(lane == 0, wx1, 0.0)
            krow = jnp.where(lane == 1, wy1, krow)
            krow = jnp.where(lane == 2, wx2, krow)
            krow = jnp.where(lane == 3, wy2, krow)
            krow = jnp.where(lane == 4, bs, krow)
            kept_v[pl.ds(it * 16, 16)] = krow

        return carry

    lax.fori_loop(0, MAX_DETS, body, jnp.int32(0))

    @pl.when(wid == 0)
    def _emit():
        pltpu.sync_copy(kept_v, out_hbm)


def _make_nms_call():
    return pl.kernel(
        _nms_body,
        mesh=plsc.VectorSubcoreMesh(
            core_axis_name="c", subcore_axis_name="s", num_cores=1),
        out_type=jax.ShapeDtypeStruct((MAX_DETS * 16,), jnp.float32),
        scratch_types=[
            pltpu.VMEM((NPER,), jnp.float32),
            pltpu.VMEM((NPER,), jnp.float32),
            pltpu.VMEM((NPER,), jnp.float32),
            pltpu.VMEM((NPER,), jnp.float32),
            pltpu.VMEM((NPER,), jnp.float32),
            pltpu.VMEM((NPER,), jnp.float32),
            pltpu.VMEM((16,), jnp.float32),
            pltpu.VMEM_SHARED((NSUB * 16,), jnp.float32),
            pltpu.VMEM((NSUB * 16,), jnp.float32),
            pltpu.VMEM((MAX_DETS * 16,), jnp.float32),
            pltpu.VMEM((16,), jnp.float32),
        ],
    )


def kernel(act, W_loc, b_loc, W_conf, b_conf, dboxes_cxywh):
    xflat = act.reshape(128, P)
    xpad = jnp.zeros((128, XCOLS), jnp.float32).at[:, PADL:PADL + P].set(xflat)
    w9 = jnp.concatenate([W_loc, W_conf], axis=0)
    w9 = w9.transpose(2, 3, 0, 1).reshape(9, C_OUT, 128)
    bias = jnp.concatenate([b_loc, b_conf], axis=0).reshape(C_OUT, 1)
    db = dboxes_cxywh.T.reshape(4, NSZ, P)
    data36 = pl.pallas_call(
        _conv_decode_body,
        out_shape=jax.ShapeDtypeStruct((6 * NSZ, P), jnp.float32),
    )(xpad, w9, bias, db)
    out16 = _make_nms_call()(data36.reshape(-1))
    return out16.reshape(MAX_DETS, 16)[:, :5]


# speedup vs baseline: 997.8734x; 1.0815x over previous
import jax
import jax.numpy as jnp
from jax import lax
from jax.experimental import pallas as pl
from jax.experimental.pallas import tpu as pltpu
from jax.experimental.pallas import tpu_sc as plsc

H = 64
W_DIM = 64
P = H * W_DIM
NSZ = 6
C_LOC = 4 * NSZ
C_CONF = 21 * NSZ
C_OUT = C_LOC + C_CONF
NCLS = 21
PADL = 72
XCOLS = 4352
N_ANCH = NSZ * P
SCALE_XY = 0.1
SCALE_WH = 0.2
NMS_THR = 0.1
MAX_DETS = 100

NSUB = 16
NPER = N_ANCH // NSUB
NCHUNK = NPER // 16
NEG = float("-inf")
C_IOU = NMS_THR / (1.0 + NMS_THR)
FAR = float(1e30)


def _perm(v, idx):
    return v.at[idx].get(mode="promise_in_bounds")


def _bfly_max(v, lane):
    for sh in (8, 4, 2, 1):
        v = jnp.maximum(v, _perm(v, lane ^ sh))
    return v


def _bfly_min_i32(v, lane):
    for sh in (8, 4, 2, 1):
        v = jnp.minimum(v, _perm(v, lane ^ sh))
    return v


def _conv_decode_body(xpad_ref, w9_ref, bias_ref, db_ref, out_ref):
    wcol = lax.broadcasted_iota(jnp.int32, (1, P), 1) % W_DIM
    mask_l = (wcol > 0).astype(jnp.float32)
    mask_r = (wcol < W_DIM - 1).astype(jnp.float32)
    acc = None
    for dy in range(3):
        for dx in range(3):
            k = dy * 3 + dx
            off = PADL + (dy - 1) * W_DIM + (dx - 1)
            xs = xpad_ref[:, off:off + P]
            prod = jnp.dot(w9_ref[k], xs, preferred_element_type=jnp.float32)
            if dx == 0:
                prod = prod * mask_l
            elif dx == 2:
                prod = prod * mask_r
            acc = prod if acc is None else acc + prod
    acc = acc + bias_ref[...]
    for s in range(NSZ):
        lx = acc[0 + s:1 + s]
        ly = acc[6 + s:7 + s]
        lw = acc[12 + s:13 + s]
        lh = acc[18 + s:19 + s]
        dcx = db_ref[0, s:s + 1]
        dcy = db_ref[1, s:s + 1]
        dwd = db_ref[2, s:s + 1]
        dht = db_ref[3, s:s + 1]
        wv = jnp.exp(lw * SCALE_WH) * dcx
        hv = jnp.exp(lh * SCALE_WH) * dcy
        cx = lx * SCALE_XY * dwd + dcx
        cy = ly * SCALE_XY * dht + dcy
        x1 = cx - wv / 2
        y1 = cy - hv / 2
        x2 = cx + wv / 2
        y2 = cy + hv / 2
        m = acc[C_LOC + s:C_LOC + s + 1]
        for cls in range(1, NCLS):
            c = C_LOC + cls * NSZ + s
            m = jnp.maximum(m, acc[c:c + 1])
        area = (x2 - x1) * (y2 - y1)
        out_ref[0 * NSZ + s:0 * NSZ + s + 1, :] = m
        out_ref[1 * NSZ + s:1 * NSZ + s + 1, :] = x1
        out_ref[2 * NSZ + s:2 * NSZ + s + 1, :] = y1
        out_ref[3 * NSZ + s:3 * NSZ + s + 1, :] = x2
        out_ref[4 * NSZ + s:4 * NSZ + s + 1, :] = y2
        out_ref[5 * NSZ + s:5 * NSZ + s + 1, :] = area


def _nms_body(data_hbm, out_hbm, sc_v, x1_v, y1_v, x2_v, y2_v, ar_v,
              pub_v, tab_sh, tab_v, kept_v):
    wid = lax.axis_index("s")
    base = wid * NPER
    pltpu.sync_copy(data_hbm.at[pl.ds(0 * N_ANCH + base, NPER)], sc_v)
    pltpu.sync_copy(data_hbm.at[pl.ds(1 * N_ANCH + base, NPER)], x1_v)
    pltpu.sync_copy(data_hbm.at[pl.ds(2 * N_ANCH + base, NPER)], y1_v)
    pltpu.sync_copy(data_hbm.at[pl.ds(3 * N_ANCH + base, NPER)], x2_v)
    pltpu.sync_copy(data_hbm.at[pl.ds(4 * N_ANCH + base, NPER)], y2_v)
    pltpu.sync_copy(data_hbm.at[pl.ds(5 * N_ANCH + base, NPER)], ar_v)
    lane = lax.broadcasted_iota(jnp.int32, (16,), 0)
    zv = jnp.zeros((16,), jnp.float32)
    for ch in range(NCHUNK):
        sl = pl.ds(ch * 16, 16)
        ar_v[sl] = ar_v[sl] * C_IOU

    @pl.when(wid == 0)
    def _zero_out():
        for r in range(MAX_DETS):
            kept_v[pl.ds(r * 16, 16)] = zv

    def body(it, carry):
        wx1, wy1, wx2, wy2, cwa = carry
        vbest = jnp.full((16,), NEG, jnp.float32)
        vcid = jnp.zeros((16,), jnp.int32)
        for ch in range(NCHUNK):
            sl = pl.ds(ch * 16, 16)
            xx1 = jnp.maximum(x1_v[sl], wx1)
            yy1 = jnp.maximum(y1_v[sl], wy1)
            xx2 = jnp.minimum(x2_v[sl], wx2)
            yy2 = jnp.minimum(y2_v[sl], wy2)
            iw = jnp.maximum(xx2 - xx1, 0.0)
            ih = jnp.maximum(yy2 - yy1, 0.0)
            inter = iw * ih
            svec = jnp.where(inter > cwa + ar_v[sl], NEG, sc_v[sl])
            sc_v[sl] = svec
            upd = svec > vbest
            vbest = jnp.where(upd, svec, vbest)
            vcid = jnp.where(upd, ch, vcid)
        mv = _bfly_max(vbest, lane)
        m = mv[0]
        cand = jnp.where(vbest == mv, vcid * 16 + lane, jnp.int32(2 ** 30))
        liv = _bfly_min_i32(cand, lane)
        li = liv[0]
        gi = (base + li).astype(jnp.float32)
        lst = (li // 16) * 16
        lln = liv - lst
        bx1 = _perm(x1_v[pl.ds(lst, 16)], lln)[0]
        by1 = _perm(y1_v[pl.ds(lst, 16)], lln)[0]
        bx2 = _perm(x2_v[pl.ds(lst, 16)], lln)[0]
        by2 = _perm(y2_v[pl.ds(lst, 16)], lln)[0]
        bar = _perm(ar_v[pl.ds(lst, 16)], lln)[0]
        row = jnp.where(lane == 0, m, 0.0)
        row = jnp.where(lane == 1, gi, row)
        row = jnp.where(lane == 2, bx1, row)
        row = jnp.where(lane == 3, by1, row)
        row = jnp.where(lane == 4, bx2, row)
        row = jnp.where(lane == 5, by2, row)
        row = jnp.where(lane == 6, bar, row)
        pub_v[...] = row
        off = (it % 2) * (NSUB * 16)
        pltpu.sync_copy(pub_v, tab_sh.at[pl.ds(off + wid * 16, 16)])
        plsc.subcore_barrier()
        pltpu.sync_copy(tab_sh.at[pl.ds(off, NSUB * 16)], tab_v)
        r0 = tab_v[pl.ds(0, 16)]
        bs, bi = r0[0], r0[1]
        wrow = r0
        for w in range(1, NSUB):
            rw = tab_v[pl.ds(w * 16, 16)]
            s_w, i_w = rw[0], rw[1]
            better = (s_w > bs) | ((s_w == bs) & (i_w < bi))
            bs = jnp.where(better, s_w, bs)
            bi = jnp.where(better, i_w, bi)
            wrow = jnp.where(better, rw, wrow)
        alive2 = bs > NEG
        nx1 = jnp.where(alive2, wrow[2], FAR)
        ny1 = jnp.where(alive2, wrow[3], FAR)
        nx2 = jnp.where(alive2, wrow[4], FAR)
        ny2 = jnp.where(alive2, wrow[5], FAR)
        ncwa = jnp.where(alive2, wrow[6], 0.0)

        @pl.when(alive2 & (wid == 0))
        def _record():
            krow = jnp.where(lane == 0, wrow[2], 0.0)
            krow = jnp.where(lane == 1, wrow[3], krow)
            krow = jnp.where(lane == 2, wrow[4], krow)
            krow = jnp.where(lane == 3, wrow[5], krow)
            krow = jnp.where(lane == 4, bs, krow)
            kept_v[pl.ds(it * 16, 16)] = krow

        return (nx1, ny1, nx2, ny2, ncwa)

    init = (jnp.float32(FAR), jnp.float32(FAR), jnp.float32(FAR),
            jnp.float32(FAR), jnp.float32(0.0))
    lax.fori_loop(0, MAX_DETS, body, init)

    @pl.when(wid == 0)
    def _emit():
        pltpu.sync_copy(kept_v, out_hbm)


def _make_nms_call():
    return pl.kernel(
        _nms_body,
        mesh=plsc.VectorSubcoreMesh(
            core_axis_name="c", subcore_axis_name="s", num_cores=1),
        out_type=jax.ShapeDtypeStruct((MAX_DETS * 16,), jnp.float32),
        scratch_types=[
            pltpu.VMEM((NPER,), jnp.float32),
            pltpu.VMEM((NPER,), jnp.float32),
            pltpu.VMEM((NPER,), jnp.float32),
            pltpu.VMEM((NPER,), jnp.float32),
            pltpu.VMEM((NPER,), jnp.float32),
            pltpu.VMEM((NPER,), jnp.float32),
            pltpu.VMEM((16,), jnp.float32),
            pltpu.VMEM_SHARED((2 * NSUB * 16,), jnp.float32),
            pltpu.VMEM((NSUB * 16,), jnp.float32),
            pltpu.VMEM((MAX_DETS * 16,), jnp.float32),
        ],
    )


def kernel(act, W_loc, b_loc, W_conf, b_conf, dboxes_cxywh):
    xflat = act.reshape(128, P)
    xpad = jnp.zeros((128, XCOLS), jnp.float32).at[:, PADL:PADL + P].set(xflat)
    w9 = jnp.concatenate([W_loc, W_conf], axis=0)
    w9 = w9.transpose(2, 3, 0, 1).reshape(9, C_OUT, 128)
    bias = jnp.concatenate([b_loc, b_conf], axis=0).reshape(C_OUT, 1)
    db = dboxes_cxywh.T.reshape(4, NSZ, P)
    data36 = pl.pallas_call(
        _conv_decode_body,
        out_shape=jax.ShapeDtypeStruct((6 * NSZ, P), jnp.float32),
    )(xpad, w9, bias, db)
    out16 = _make_nms_call()(data36.reshape(-1))
    return out16.reshape(MAX_DETS, 16)[:, :5]


# speedup vs baseline: 1060.7398x; 1.0630x over previous
import jax
import jax.numpy as jnp
from jax import lax
from jax.experimental import pallas as pl
from jax.experimental.pallas import tpu as pltpu
from jax.experimental.pallas import tpu_sc as plsc

H = 64
W_DIM = 64
P = H * W_DIM
NSZ = 6
C_LOC = 4 * NSZ
C_CONF = 21 * NSZ
C_OUT = C_LOC + C_CONF
NCLS = 21
PADL = 72
XCOLS = 4352
N_ANCH = NSZ * P
SCALE_XY = 0.1
SCALE_WH = 0.2
NMS_THR = 0.1
MAX_DETS = 100

NSUB = 16
NPER = N_ANCH // NSUB
NCHUNK = NPER // 16
NEG = float("-inf")
C_IOU = NMS_THR / (1.0 + NMS_THR)
FAR = float(1e30)


def _perm(v, idx):
    return v.at[idx].get(mode="promise_in_bounds")


def _bfly_max(v, lane):
    for sh in (8, 4, 2, 1):
        v = jnp.maximum(v, _perm(v, lane ^ sh))
    return v


def _bfly_min_i32(v, lane):
    for sh in (8, 4, 2, 1):
        v = jnp.minimum(v, _perm(v, lane ^ sh))
    return v


def _bfly_min_f32(v, lane):
    for sh in (8, 4, 2, 1):
        v = jnp.minimum(v, _perm(v, lane ^ sh))
    return v


def _conv_decode_body(xpad_ref, w9_ref, bias_ref, db_ref, out_ref):
    wcol = lax.broadcasted_iota(jnp.int32, (1, P), 1) % W_DIM
    mask_l = (wcol > 0).astype(jnp.float32)
    mask_r = (wcol < W_DIM - 1).astype(jnp.float32)
    acc = None
    for dy in range(3):
        for dx in range(3):
            k = dy * 3 + dx
            off = PADL + (dy - 1) * W_DIM + (dx - 1)
            xs = xpad_ref[:, off:off + P]
            prod = jnp.dot(w9_ref[k], xs, preferred_element_type=jnp.float32)
            if dx == 0:
                prod = prod * mask_l
            elif dx == 2:
                prod = prod * mask_r
            acc = prod if acc is None else acc + prod
    acc = acc + bias_ref[...]
    for s in range(NSZ):
        lx = acc[0 + s:1 + s]
        ly = acc[6 + s:7 + s]
        lw = acc[12 + s:13 + s]
        lh = acc[18 + s:19 + s]
        dcx = db_ref[0, s:s + 1]
        dcy = db_ref[1, s:s + 1]
        dwd = db_ref[2, s:s + 1]
        dht = db_ref[3, s:s + 1]
        wv = jnp.exp(lw * SCALE_WH) * dcx
        hv = jnp.exp(lh * SCALE_WH) * dcy
        cx = lx * SCALE_XY * dwd + dcx
        cy = ly * SCALE_XY * dht + dcy
        x1 = cx - wv / 2
        y1 = cy - hv / 2
        x2 = cx + wv / 2
        y2 = cy + hv / 2
        m = acc[C_LOC + s:C_LOC + s + 1]
        for cls in range(1, NCLS):
            c = C_LOC + cls * NSZ + s
            m = jnp.maximum(m, acc[c:c + 1])
        area = (x2 - x1) * (y2 - y1)
        out_ref[0 * NSZ + s:0 * NSZ + s + 1, :] = m
        out_ref[1 * NSZ + s:1 * NSZ + s + 1, :] = x1
        out_ref[2 * NSZ + s:2 * NSZ + s + 1, :] = y1
        out_ref[3 * NSZ + s:3 * NSZ + s + 1, :] = x2
        out_ref[4 * NSZ + s:4 * NSZ + s + 1, :] = y2
        out_ref[5 * NSZ + s:5 * NSZ + s + 1, :] = area


def _nms_body(data_hbm, out_hbm, sc_v, x1_v, y1_v, x2_v, y2_v, ar_v,
              pub_v, tab_sh, tab_v, kept_v):
    wid = lax.axis_index("s")
    base = wid * NPER
    pltpu.sync_copy(data_hbm.at[pl.ds(0 * N_ANCH + base, NPER)], sc_v)
    pltpu.sync_copy(data_hbm.at[pl.ds(1 * N_ANCH + base, NPER)], x1_v)
    pltpu.sync_copy(data_hbm.at[pl.ds(2 * N_ANCH + base, NPER)], y1_v)
    pltpu.sync_copy(data_hbm.at[pl.ds(3 * N_ANCH + base, NPER)], x2_v)
    pltpu.sync_copy(data_hbm.at[pl.ds(4 * N_ANCH + base, NPER)], y2_v)
    pltpu.sync_copy(data_hbm.at[pl.ds(5 * N_ANCH + base, NPER)], ar_v)
    lane = lax.broadcasted_iota(jnp.int32, (16,), 0)
    zv = jnp.zeros((16,), jnp.float32)
    for ch in range(NCHUNK):
        sl = pl.ds(ch * 16, 16)
        ar_v[sl] = ar_v[sl] * C_IOU

    @pl.when(wid == 0)
    def _zero_out():
        for r in range(MAX_DETS):
            kept_v[pl.ds(r * 16, 16)] = zv

    def body(it, carry):
        wx1, wy1, wx2, wy2, cwa = carry
        acc = [[jnp.full((16,), NEG, jnp.float32), jnp.zeros((16,), jnp.int32)]
               for _ in range(2)]
        for ch in range(NCHUNK):
            sl = pl.ds(ch * 16, 16)
            xx1 = jnp.maximum(x1_v[sl], wx1)
            yy1 = jnp.maximum(y1_v[sl], wy1)
            xx2 = jnp.minimum(x2_v[sl], wx2)
            yy2 = jnp.minimum(y2_v[sl], wy2)
            iw = jnp.maximum(xx2 - xx1, 0.0)
            ih = jnp.maximum(yy2 - yy1, 0.0)
            inter = iw * ih
            svec = jnp.where(inter > cwa + ar_v[sl], NEG, sc_v[sl])
            sc_v[sl] = svec
            vb, vc = acc[ch % 2]
            upd = svec > vb
            acc[ch % 2][0] = jnp.where(upd, svec, vb)
            acc[ch % 2][1] = jnp.where(upd, ch, vc)
        mrg = (acc[1][0] > acc[0][0]) | (
            (acc[1][0] == acc[0][0]) & (acc[1][1] < acc[0][1]))
        vbest = jnp.where(mrg, acc[1][0], acc[0][0])
        vcid = jnp.where(mrg, acc[1][1], acc[0][1])
        mv = _bfly_max(vbest, lane)
        m = mv[0]
        cand = jnp.where(vbest == mv, vcid * 16 + lane, jnp.int32(2 ** 30))
        liv = _bfly_min_i32(cand, lane)
        li = liv[0]
        gi = (base + li).astype(jnp.float32)
        lst = (li // 16) * 16
        lln = liv - lst
        bx1 = _perm(x1_v[pl.ds(lst, 16)], lln)[0]
        by1 = _perm(y1_v[pl.ds(lst, 16)], lln)[0]
        bx2 = _perm(x2_v[pl.ds(lst, 16)], lln)[0]
        by2 = _perm(y2_v[pl.ds(lst, 16)], lln)[0]
        bar = _perm(ar_v[pl.ds(lst, 16)], lln)[0]
        row = jnp.where(lane == 0, m, 0.0)
        row = jnp.where(lane == 1, gi, row)
        row = jnp.where(lane == 2, bx1, row)
        row = jnp.where(lane == 3, by1, row)
        row = jnp.where(lane == 4, bx2, row)
        row = jnp.where(lane == 5, by2, row)
        row = jnp.where(lane == 6, bar, row)
        pub_v[...] = row
        off = (it % 2) * (NSUB * 16)
        pltpu.sync_copy(pub_v, tab_sh.at[pl.ds(off + wid * 16, 16)])
        plsc.subcore_barrier()
        pltpu.sync_copy(tab_sh.at[pl.ds(off, NSUB * 16)], tab_v)
        def comb(a, b):
            sa, ia, ra = a
            sb, ib, rb = b
            better = (sb > sa) | ((sb == sa) & (ib < ia))
            return (jnp.where(better, sb, sa), jnp.where(better, ib, ia),
                    jnp.where(better, rb, ra))

        nodes = []
        for w in range(NSUB):
            rw = tab_v[pl.ds(w * 16, 16)]
            nodes.append((rw[0], rw[1], rw))
        while len(nodes) > 1:
            nodes = [comb(nodes[i], nodes[i + 1])
                     for i in range(0, len(nodes), 2)]
        bs, _, wrow = nodes[0]
        alive2 = bs > NEG
        nx1 = jnp.where(alive2, wrow[2], FAR)
        ny1 = jnp.where(alive2, wrow[3], FAR)
        nx2 = jnp.where(alive2, wrow[4], FAR)
        ny2 = jnp.where(alive2, wrow[5], FAR)
        ncwa = jnp.where(alive2, wrow[6], 0.0)

        @pl.when(alive2 & (wid == 0))
        def _record():
            krow = jnp.where(lane == 0, wrow[2], 0.0)
            krow = jnp.where(lane == 1, wrow[3], krow)
            krow = jnp.where(lane == 2, wrow[4], krow)
            krow = jnp.where(lane == 3, wrow[5], krow)
            krow = jnp.where(lane == 4, bs, krow)
            kept_v[pl.ds(it * 16, 16)] = krow

        return (nx1, ny1, nx2, ny2, ncwa)

    init = (jnp.float32(FAR), jnp.float32(FAR), jnp.float32(FAR),
            jnp.float32(FAR), jnp.float32(0.0))
    lax.fori_loop(0, MAX_DETS, body, init)

    @pl.when(wid == 0)
    def _emit():
        pltpu.sync_copy(kept_v, out_hbm)


def _make_nms_call():
    return pl.kernel(
        _nms_body,
        mesh=plsc.VectorSubcoreMesh(
            core_axis_name="c", subcore_axis_name="s", num_cores=1),
        out_type=jax.ShapeDtypeStruct((MAX_DETS * 16,), jnp.float32),
        scratch_types=[
            pltpu.VMEM((NPER,), jnp.float32),
            pltpu.VMEM((NPER,), jnp.float32),
            pltpu.VMEM((NPER,), jnp.float32),
            pltpu.VMEM((NPER,), jnp.float32),
            pltpu.VMEM((NPER,), jnp.float32),
            pltpu.VMEM((NPER,), jnp.float32),
            pltpu.VMEM((16,), jnp.float32),
            pltpu.VMEM_SHARED((2 * NSUB * 16,), jnp.float32),
            pltpu.VMEM((NSUB * 16,), jnp.float32),
            pltpu.VMEM((MAX_DETS * 16,), jnp.float32),
        ],
    )


def kernel(act, W_loc, b_loc, W_conf, b_conf, dboxes_cxywh):
    xflat = act.reshape(128, P)
    xpad = jnp.zeros((128, XCOLS), jnp.float32).at[:, PADL:PADL + P].set(xflat)
    w9 = jnp.concatenate([W_loc, W_conf], axis=0)
    w9 = w9.transpose(2, 3, 0, 1).reshape(9, C_OUT, 128)
    bias = jnp.concatenate([b_loc, b_conf], axis=0).reshape(C_OUT, 1)
    db = dboxes_cxywh.T.reshape(4, NSZ, P)
    data36 = pl.pallas_call(
        _conv_decode_body,
        out_shape=jax.ShapeDtypeStruct((6 * NSZ, P), jnp.float32),
    )(xpad, w9, bias, db)
    out16 = _make_nms_call()(data36.reshape(-1))
    return out16.reshape(MAX_DETS, 16)[:, :5]
